# baseline (speedup 1.0000x reference)
import functools

import jax
import jax.numpy as jnp
from jax import lax
from jax.experimental import pallas as pl
from jax.experimental.pallas import tpu as pltpu
from jax.experimental.pallas import tpu_sc as plsc

F = 16
NC_TOTAL = 32
SUB = 128
NSUBCH = 4
EB = NSUBCH * SUB
NSUB = 16
LANES = 16
NTAP = 3

_SPLAT_DNUMS = lax.GatherDimensionNumbers(
    offset_dims=(), collapsed_slice_dims=(0,), start_index_map=(0,))


def _lane_splat(v, lane):
    idx = jnp.full((LANES, 1), lane, jnp.int32)
    return lax.gather(v, idx, _SPLAT_DNUMS, (1,),
                      mode=lax.GatherScatterMode.PROMISE_IN_BOUNDS)


def _layer_sc(x_flat, src_r, dst_r, w_r, np_, bpt, rpt):
    n2 = 2 * np_
    mesh = plsc.VectorSubcoreMesh(core_axis_name="c", subcore_axis_name="s")

    @functools.partial(
        pl.kernel,
        mesh=mesh,
        compiler_params=pltpu.CompilerParams(use_tc_tiling_on_sc=False),
        out_type=jax.ShapeDtypeStruct((NTAP * n2, F), jnp.float32),
        scratch_types=[
            pltpu.VMEM((3, NSUBCH, SUB), jnp.int32),
            pltpu.VMEM((3, NSUBCH, SUB), jnp.int32),
            pltpu.VMEM((3, NSUBCH, SUB), jnp.float32),
            pltpu.VMEM((2, NSUBCH * SUB, F), jnp.float32),
            pltpu.VMEM_SHARED((np_, F), jnp.float32),
            pltpu.SemaphoreType.DMA,
            pltpu.SemaphoreType.DMA,
            pltpu.SemaphoreType.DMA,
        ],
    )
    def k(x_hbm, src_hbm, dst_hbm, w_hbm, y_hbm, ts_src, ts_dst, ts_w, rows,
          acc, g_sem, s_sem, i_sem):
        c = lax.axis_index("c")
        s = lax.axis_index("s")
        base = s * rpt
        nfull = rpt // SUB
        rem = rpt - nfull * SUB

        def rslot(p, j):
            return rows.at[p].at[pl.ds(j * SUB, SUB)]

        def issue_idx(b, p3):
            pltpu.async_copy(src_hbm.at[s, b], ts_src.at[p3], i_sem)
            pltpu.async_copy(dst_hbm.at[s, b], ts_dst.at[p3], i_sem)
            pltpu.async_copy(w_hbm.at[s, b], ts_w.at[p3], i_sem)

        def wait_idx(p3):
            for ref in (ts_src.at[p3], ts_dst.at[p3]):
                pltpu.make_async_copy(dst_hbm.at[s, 0], ref, i_sem).wait()
            pltpu.make_async_copy(w_hbm.at[s, 0], ts_w.at[p3], i_sem).wait()

        def adjust_src(p3, off):
            for j in range(NSUBCH):
                for g in range(SUB // LANES):
                    sl = pl.ds(g * LANES, LANES)
                    ts_src[p3, j, sl] = ts_src[p3, j, sl] + off

        def issue_gathers(t, p2, p3):
            for j in range(NSUBCH):
                @pl.when(t == 0)
                def _():
                    pltpu.async_copy(x_hbm.at[ts_src.at[p3, j]],
                                     rslot(p2, j), g_sem)

                @pl.when(t > 0)
                def _():
                    pltpu.async_copy(y_hbm.at[ts_src.at[p3, j]],
                                     rslot(p2, j), g_sem)

        def drain_rows(p2, sem):
            for j in range(NSUBCH):
                pltpu.make_async_copy(x_hbm.at[pl.ds(0, SUB)], rslot(p2, j),
                                      sem).wait()

        def scale_scatter(p2, p3):
            for j in range(NSUBCH):
                slot = rslot(p2, j)
                for g in range(SUB // LANES):
                    e0 = g * LANES
                    wv = ts_w[p3, j, pl.ds(e0, LANES)]
                    for l in range(LANES):
                        spl = _lane_splat(wv, l)
                        slot[e0 + l, :] = slot[e0 + l, :] * spl
                pltpu.async_copy(slot, acc.at[ts_dst.at[p3, j]], s_sem,
                                 add=True)

        def phase_body(t, _):
            off = c * np_ + jnp.where(t > 0, (t - 1) * n2, 0)

            def _zrow(e2, _2):
                rows[0, e2, :] = jnp.zeros((F,), jnp.float32)
                return 0
            lax.fori_loop(0, SUB, _zrow, 0)

            def _zcp(r, _2):
                pltpu.sync_copy(rows.at[0].at[pl.ds(0, SUB)],
                                acc.at[pl.ds(base + r * SUB, SUB)])
                return 0
            lax.fori_loop(0, nfull, _zcp, 0)
            if rem:
                pltpu.sync_copy(rows.at[0].at[pl.ds(0, rem)],
                                acc.at[pl.ds(base + nfull * SUB, rem)])
            plsc.subcore_barrier()

            issue_idx(0, 0)
            wait_idx(0)
            adjust_src(0, off)
            if bpt > 1:
                issue_idx(1, 1)
            issue_gathers(t, 0, 0)

            def blk_body(b, _2):
                p2 = lax.rem(b, 2)
                q2 = lax.rem(b + 1, 2)
                p3 = lax.rem(b, 3)
                drain_rows(p2, g_sem)

                @pl.when(b < bpt - 1)
                def _():
                    q3 = lax.rem(b + 1, 3)

                    @pl.when(b >= 1)
                    def _():
                        drain_rows(q2, s_sem)
                    wait_idx(q3)
                    adjust_src(q3, off)
                    issue_gathers(t, q2, q3)

                    @pl.when(b < bpt - 2)
                    def _():
                        issue_idx(b + 2, lax.rem(b + 2, 3))
                scale_scatter(p2, p3)
                return 0
            lax.fori_loop(0, bpt, blk_body, 0)
            if bpt > 1:
                drain_rows((bpt - 2) % 2, s_sem)
            drain_rows((bpt - 1) % 2, s_sem)
            plsc.subcore_barrier()

            pltpu.sync_copy(acc.at[pl.ds(base, rpt)],
                            y_hbm.at[pl.ds(t * n2 + c * np_ + base, rpt)])
            plsc.subcore_barrier()
            return 0
        lax.fori_loop(0, NTAP, phase_body, 0)

    return k(x_flat, src_r, dst_r, w_r)


def _readin_tc(state, big_w, big_b, n, np_, bn):
    def body(s_ref, w_ref, b_ref, o_ref):
        hb = jnp.dot(s_ref[...], w_ref[...],
                     preferred_element_type=jnp.float32) + b_ref[...]
        o_ref[0] = hb[:, :F]
        o_ref[1] = hb[:, F:]

    return pl.pallas_call(
        body,
        grid=(n // bn,),
        in_specs=[
            pl.BlockSpec((bn, state.shape[1]), lambda i: (i, 0)),
            pl.BlockSpec(big_w.shape, lambda i: (0, 0)),
            pl.BlockSpec(big_b.shape, lambda i: (0, 0)),
        ],
        out_specs=pl.BlockSpec((2, bn, F), lambda i: (0, i, 0)),
        out_shape=jax.ShapeDtypeStruct((2, np_, F), jnp.float32),
    )(state, big_w, big_b)


def _cat_taps(ya_ref, h_ref, w_ref, b_ref):
    hv = h_ref[...]
    yv = ya_ref[...]
    h_cat = jnp.concatenate([hv[0], hv[1]], axis=-1)
    acc = jnp.dot(h_cat, w_ref[0], preferred_element_type=jnp.float32)
    for kk in range(NTAP):
        y_cat = jnp.concatenate([yv[kk, 0], yv[kk, 1]], axis=-1)
        acc = acc + jnp.dot(y_cat, w_ref[kk + 1],
                            preferred_element_type=jnp.float32)
    acc = acc + b_ref[...]
    return jnp.where(acc >= 0, acc, 0.01 * acc)


def _combine_tc(h2, ya, taps_w, taps_b, n, np_, bn):
    def body(h_ref, ya_ref, w_ref, b_ref, o_ref):
        r = _cat_taps(ya_ref, h_ref, w_ref, b_ref)
        o_ref[0] = r[:, :F]
        o_ref[1] = r[:, F:]

    spec2 = pl.BlockSpec((2, bn, F), lambda i: (0, i, 0))
    return pl.pallas_call(
        body,
        grid=(n // bn,),
        in_specs=[
            spec2,
            pl.BlockSpec((NTAP, 2, bn, F), lambda i: (0, 0, i, 0)),
            pl.BlockSpec(taps_w.shape, lambda i: (0, 0, 0)),
            pl.BlockSpec(taps_b.shape, lambda i: (0, 0)),
        ],
        out_specs=spec2,
        out_shape=jax.ShapeDtypeStruct((2, np_, F), jnp.float32),
    )(h2, ya, taps_w, taps_b)


def _combine_readout_tc(h2, ya, taps_w, taps_b, ro_w, ro_b, n, bn):
    def body(h_ref, ya_ref, w_ref, b_ref, rw_ref, rb_ref, o_ref):
        r = _cat_taps(ya_ref, h_ref, w_ref, b_ref)
        o_ref[...] = jnp.dot(r, rw_ref[...],
                             preferred_element_type=jnp.float32) + rb_ref[...]

    spec2 = pl.BlockSpec((2, bn, F), lambda i: (0, i, 0))
    out_dim = ro_w.shape[1]
    return pl.pallas_call(
        body,
        grid=(n // bn,),
        in_specs=[
            spec2,
            pl.BlockSpec((NTAP, 2, bn, F), lambda i: (0, 0, i, 0)),
            pl.BlockSpec(taps_w.shape, lambda i: (0, 0, 0)),
            pl.BlockSpec(taps_b.shape, lambda i: (0, 0)),
            pl.BlockSpec(ro_w.shape, lambda i: (0, 0)),
            pl.BlockSpec(ro_b.shape, lambda i: (0, 0)),
        ],
        out_specs=pl.BlockSpec((bn, out_dim), lambda i: (i, 0)),
        out_shape=jax.ShapeDtypeStruct((n, out_dim), jnp.float32),
    )(h2, ya, taps_w, taps_b, ro_w, ro_b)


def kernel(state, edge_index, edge_attr, phi_W, phi_b, rho_W, rho_b,
           readin_W, readin_b, taps1_W, taps1_b, taps2_W, taps2_b,
           readout_W, readout_b):
    n = state.shape[0]
    e = edge_index.shape[1]
    state_dim = state.shape[1]
    n_obs_pairs = (state_dim - 10) // 2

    tail_w = (phi_W @ rho_W) @ readin_W[10:12]
    big_w = jnp.concatenate(
        [readin_W[:10], jnp.tile(tail_w, (n_obs_pairs, 1))], axis=0)
    big_b = ((n_obs_pairs * phi_b) @ rho_W + rho_b) @ readin_W[10:12] + readin_b
    big_b = big_b.reshape(1, NC_TOTAL)

    per_tile = -(-e // NSUB)
    bpt = -(-per_tile // EB)
    e_pad = NSUB * bpt * EB
    pad = e_pad - e
    src = edge_index[0]
    dst = edge_index[1]
    w = edge_attr[:, 0]
    rpt = ((-(-n // NSUB)) + 7) // 8 * 8
    np_ = NSUB * rpt
    src_p = jnp.concatenate([src, jnp.zeros((pad,), jnp.int32)])
    dst_p = jnp.concatenate([dst, jnp.zeros((pad,), jnp.int32)])
    w_p = jnp.concatenate([w, jnp.zeros((pad,), jnp.float32)])
    src_r = src_p.reshape(NSUB, bpt, NSUBCH, SUB)
    dst_r = dst_p.reshape(NSUB, bpt, NSUBCH, SUB)
    w_r = w_p.reshape(NSUB, bpt, NSUBCH, SUB)

    bn = 4000
    assert n % bn == 0

    h2 = _readin_tc(state, big_w, big_b, n, np_, bn)

    b1 = taps1_b.reshape(1, NC_TOTAL)
    b2 = taps2_b.reshape(1, NC_TOTAL)
    ro_b = readout_b.reshape(1, readout_W.shape[1])

    ya = _layer_sc(h2.reshape(2 * np_, F), src_r, dst_r, w_r, np_, bpt, rpt)
    h2 = _combine_tc(h2, ya.reshape(NTAP, 2, np_, F), taps1_W, b1, n, np_, bn)

    ya = _layer_sc(h2.reshape(2 * np_, F), src_r, dst_r, w_r, np_, bpt, rpt)
    out = _combine_readout_tc(h2, ya.reshape(NTAP, 2, np_, F), taps2_W, b2,
                              readout_W, ro_b, n, bn)
    return out


# speedup vs baseline: 1.1357x; 1.1357x over previous
import functools

import jax
import jax.numpy as jnp
from jax import lax
from jax.experimental import pallas as pl
from jax.experimental.pallas import tpu as pltpu
from jax.experimental.pallas import tpu_sc as plsc

F = 16
NC_TOTAL = 32
SUB = 128
RING = 8
GAHEAD = 5
NSUB = 16
LANES = 16
NTAP = 3

_SPLAT_DNUMS = lax.GatherDimensionNumbers(
    offset_dims=(), collapsed_slice_dims=(0,), start_index_map=(0,))


def _lane_splat(v, lane):
    idx = jnp.full((LANES, 1), lane, jnp.int32)
    return lax.gather(v, idx, _SPLAT_DNUMS, (1,),
                      mode=lax.GatherScatterMode.PROMISE_IN_BOUNDS)


def _layer_sc(x_flat, src_r, dst_r, w_r, np_, bpt, rpt):
    n2 = 2 * np_
    nsup = bpt // RING
    mesh = plsc.VectorSubcoreMesh(core_axis_name="c", subcore_axis_name="s")

    @functools.partial(
        pl.kernel,
        mesh=mesh,
        compiler_params=pltpu.CompilerParams(use_tc_tiling_on_sc=False),
        out_type=jax.ShapeDtypeStruct((NTAP * n2, F), jnp.float32),
        scratch_types=[
            pltpu.VMEM((RING, SUB), jnp.int32),
            pltpu.VMEM((RING, SUB), jnp.int32),
            pltpu.VMEM((RING, SUB), jnp.float32),
            pltpu.VMEM((RING * SUB, F), jnp.float32),
            pltpu.VMEM_SHARED((np_, F), jnp.float32),
        ] + [pltpu.SemaphoreType.DMA] * (3 * RING),
    )
    def k(x_hbm, src_hbm, dst_hbm, w_hbm, y_hbm, ts_src, ts_dst, ts_w, rows,
          acc, *sems):
        g_sem = sems[0:RING]
        s_sem = sems[RING:2 * RING]
        i_sem = sems[2 * RING:3 * RING]
        c = lax.axis_index("c")
        s = lax.axis_index("s")
        base = s * rpt
        nfull = rpt // SUB
        rem = rpt - nfull * SUB

        def rslot(r):
            return rows.at[pl.ds(r * SUB, SUB)]

        def issue_idx(b, r):
            pltpu.async_copy(src_hbm.at[s, b], ts_src.at[r], i_sem[r])
            pltpu.async_copy(dst_hbm.at[s, b], ts_dst.at[r], i_sem[r])
            pltpu.async_copy(w_hbm.at[s, b], ts_w.at[r], i_sem[r])

        def wait_idx(r):
            for ref in (ts_src.at[r], ts_dst.at[r]):
                pltpu.make_async_copy(dst_hbm.at[s, 0], ref, i_sem[r]).wait()
            pltpu.make_async_copy(w_hbm.at[s, 0], ts_w.at[r], i_sem[r]).wait()

        def adjust_src(r, off):
            for g in range(SUB // LANES):
                sl = pl.ds(g * LANES, LANES)
                ts_src[r, sl] = ts_src[r, sl] + off

        def issue_gather(t, r):
            @pl.when(t == 0)
            def _():
                pltpu.async_copy(x_hbm.at[ts_src.at[r]], rslot(r), g_sem[r])

            @pl.when(t > 0)
            def _():
                pltpu.async_copy(y_hbm.at[ts_src.at[r]], rslot(r), g_sem[r])

        def drain_slot(r, sem):
            pltpu.make_async_copy(x_hbm.at[pl.ds(0, SUB)], rslot(r),
                                  sem).wait()

        def scale_scatter(r):
            slot = rslot(r)
            for g in range(SUB // LANES):
                e0 = g * LANES
                wv = ts_w[r, pl.ds(e0, LANES)]
                for l in range(LANES):
                    spl = _lane_splat(wv, l)
                    slot[e0 + l, :] = slot[e0 + l, :] * spl
            pltpu.async_copy(slot, acc.at[ts_dst.at[r]], s_sem[r], add=True)

        def phase_body(t, _):
            off = c * np_ + jnp.where(t > 0, (t - 1) * n2, 0)

            def _zrow(e2, _2):
                rows[e2, :] = jnp.zeros((F,), jnp.float32)
                return 0
            lax.fori_loop(0, SUB, _zrow, 0)

            def _zcp(zr, _2):
                pltpu.sync_copy(rows.at[pl.ds(0, SUB)],
                                acc.at[pl.ds(base + zr * SUB, SUB)])
                return 0
            lax.fori_loop(0, nfull, _zcp, 0)
            if rem:
                pltpu.sync_copy(rows.at[pl.ds(0, rem)],
                                acc.at[pl.ds(base + nfull * SUB, rem)])
            plsc.subcore_barrier()

            for r in range(RING):
                issue_idx(r, r)
            for r in range(GAHEAD):
                wait_idx(r)
                adjust_src(r, off)
                issue_gather(t, r)

            def sup_body(u, _2):
                for r in range(RING):
                    b = u * RING + r
                    drain_slot(r, g_sem[r])
                    scale_scatter(r)

                    @pl.when(u < nsup - 1)
                    def _():
                        issue_idx(b + RING, r)
                    r5 = (r + GAHEAD) % RING

                    def _ahead():
                        drain_slot(r5, s_sem[r5])
                        wait_idx(r5)
                        adjust_src(r5, off)
                        issue_gather(t, r5)
                    if r < RING - GAHEAD:
                        @pl.when(u > 0)
                        def _():
                            drain_slot(r5, s_sem[r5])
                        wait_idx(r5)
                        adjust_src(r5, off)
                        issue_gather(t, r5)
                    else:
                        @pl.when(u < nsup - 1)
                        def _():
                            _ahead()
                return 0
            lax.fori_loop(0, nsup, sup_body, 0)
            for r in range(RING):
                drain_slot(r, s_sem[r])
            plsc.subcore_barrier()

            pltpu.sync_copy(acc.at[pl.ds(base, rpt)],
                            y_hbm.at[pl.ds(t * n2 + c * np_ + base, rpt)])
            plsc.subcore_barrier()
            return 0
        lax.fori_loop(0, NTAP, phase_body, 0)

    return k(x_flat, src_r, dst_r, w_r)


def _readin_tc(state, big_w, big_b, n, np_, bn):
    def body(s_ref, w_ref, b_ref, o_ref):
        hb = jnp.dot(s_ref[...], w_ref[...],
                     preferred_element_type=jnp.float32) + b_ref[...]
        o_ref[0] = hb[:, :F]
        o_ref[1] = hb[:, F:]

    return pl.pallas_call(
        body,
        grid=(n // bn,),
        in_specs=[
            pl.BlockSpec((bn, state.shape[1]), lambda i: (i, 0)),
            pl.BlockSpec(big_w.shape, lambda i: (0, 0)),
            pl.BlockSpec(big_b.shape, lambda i: (0, 0)),
        ],
        out_specs=pl.BlockSpec((2, bn, F), lambda i: (0, i, 0)),
        out_shape=jax.ShapeDtypeStruct((2, np_, F), jnp.float32),
    )(state, big_w, big_b)


def _cat_taps(ya_ref, h_ref, w_ref, b_ref):
    hv = h_ref[...]
    yv = ya_ref[...]
    h_cat = jnp.concatenate([hv[0], hv[1]], axis=-1)
    acc = jnp.dot(h_cat, w_ref[0], preferred_element_type=jnp.float32)
    for kk in range(NTAP):
        y_cat = jnp.concatenate([yv[kk, 0], yv[kk, 1]], axis=-1)
        acc = acc + jnp.dot(y_cat, w_ref[kk + 1],
                            preferred_element_type=jnp.float32)
    acc = acc + b_ref[...]
    return jnp.where(acc >= 0, acc, 0.01 * acc)


def _combine_tc(h2, ya, taps_w, taps_b, n, np_, bn):
    def body(h_ref, ya_ref, w_ref, b_ref, o_ref):
        r = _cat_taps(ya_ref, h_ref, w_ref, b_ref)
        o_ref[0] = r[:, :F]
        o_ref[1] = r[:, F:]

    spec2 = pl.BlockSpec((2, bn, F), lambda i: (0, i, 0))
    return pl.pallas_call(
        body,
        grid=(n // bn,),
        in_specs=[
            spec2,
            pl.BlockSpec((NTAP, 2, bn, F), lambda i: (0, 0, i, 0)),
            pl.BlockSpec(taps_w.shape, lambda i: (0, 0, 0)),
            pl.BlockSpec(taps_b.shape, lambda i: (0, 0)),
        ],
        out_specs=spec2,
        out_shape=jax.ShapeDtypeStruct((2, np_, F), jnp.float32),
    )(h2, ya, taps_w, taps_b)


def _combine_readout_tc(h2, ya, taps_w, taps_b, ro_w, ro_b, n, bn):
    def body(h_ref, ya_ref, w_ref, b_ref, rw_ref, rb_ref, o_ref):
        r = _cat_taps(ya_ref, h_ref, w_ref, b_ref)
        o_ref[...] = jnp.dot(r, rw_ref[...],
                             preferred_element_type=jnp.float32) + rb_ref[...]

    spec2 = pl.BlockSpec((2, bn, F), lambda i: (0, i, 0))
    out_dim = ro_w.shape[1]
    return pl.pallas_call(
        body,
        grid=(n // bn,),
        in_specs=[
            spec2,
            pl.BlockSpec((NTAP, 2, bn, F), lambda i: (0, 0, i, 0)),
            pl.BlockSpec(taps_w.shape, lambda i: (0, 0, 0)),
            pl.BlockSpec(taps_b.shape, lambda i: (0, 0)),
            pl.BlockSpec(ro_w.shape, lambda i: (0, 0)),
            pl.BlockSpec(ro_b.shape, lambda i: (0, 0)),
        ],
        out_specs=pl.BlockSpec((bn, out_dim), lambda i: (i, 0)),
        out_shape=jax.ShapeDtypeStruct((n, out_dim), jnp.float32),
    )(h2, ya, taps_w, taps_b, ro_w, ro_b)


def kernel(state, edge_index, edge_attr, phi_W, phi_b, rho_W, rho_b,
           readin_W, readin_b, taps1_W, taps1_b, taps2_W, taps2_b,
           readout_W, readout_b):
    n = state.shape[0]
    e = edge_index.shape[1]
    state_dim = state.shape[1]
    n_obs_pairs = (state_dim - 10) // 2

    tail_w = (phi_W @ rho_W) @ readin_W[10:12]
    big_w = jnp.concatenate(
        [readin_W[:10], jnp.tile(tail_w, (n_obs_pairs, 1))], axis=0)
    big_b = ((n_obs_pairs * phi_b) @ rho_W + rho_b) @ readin_W[10:12] + readin_b
    big_b = big_b.reshape(1, NC_TOTAL)

    per_tile = -(-e // NSUB)
    bpt = -(-per_tile // SUB)
    bpt = -(-bpt // RING) * RING
    e_pad = NSUB * bpt * SUB
    pad = e_pad - e
    src = edge_index[0]
    dst = edge_index[1]
    w = edge_attr[:, 0]
    rpt = ((-(-n // NSUB)) + 7) // 8 * 8
    np_ = NSUB * rpt
    src_p = jnp.concatenate([src, jnp.zeros((pad,), jnp.int32)])
    dst_p = jnp.concatenate([dst, jnp.zeros((pad,), jnp.int32)])
    w_p = jnp.concatenate([w, jnp.zeros((pad,), jnp.float32)])
    src_r = src_p.reshape(NSUB, bpt, SUB)
    dst_r = dst_p.reshape(NSUB, bpt, SUB)
    w_r = w_p.reshape(NSUB, bpt, SUB)

    bn = 4000
    assert n % bn == 0

    h2 = _readin_tc(state, big_w, big_b, n, np_, bn)

    b1 = taps1_b.reshape(1, NC_TOTAL)
    b2 = taps2_b.reshape(1, NC_TOTAL)
    ro_b = readout_b.reshape(1, readout_W.shape[1])

    ya = _layer_sc(h2.reshape(2 * np_, F), src_r, dst_r, w_r, np_, bpt, rpt)
    h2 = _combine_tc(h2, ya.reshape(NTAP, 2, np_, F), taps1_W, b1, n, np_, bn)

    ya = _layer_sc(h2.reshape(2 * np_, F), src_r, dst_r, w_r, np_, bpt, rpt)
    out = _combine_readout_tc(h2, ya.reshape(NTAP, 2, np_, F), taps2_W, b2,
                              readout_W, ro_b, n, bn)
    return out
